# Initial kernel scaffold; baseline (speedup 1.0000x reference)
import jax
import jax.numpy as jnp
from jax.experimental import pallas as pl


def kernel(output, target, one_hot):
    raise NotImplementedError("write your pallas kernel here")


# speedup vs baseline: 9.9961x; 9.9961x over previous
import functools
import math

import jax
import jax.numpy as jnp
from jax.experimental import pallas as pl

LABEL_SMOOTHING = 0.1
IGNORE_INDEX = -100
ROW_BLOCK = 128


def _loss_body(x_ref, t_ref, o_ref, *, V, B, RB):
    s = LABEL_SMOOTHING / (V - 2)
    c = 1.0 - LABEL_SMOOTHING
    Z = V + IGNORE_INDEX
    kc = (V - 2) * s * math.log(s) + c * math.log(c)
    s_log_s = s * math.log(s)

    i = pl.program_id(0)
    x = x_ref[...]
    t = t_ref[0]
    m = jnp.max(x, axis=1, keepdims=True)
    se = jnp.sum(jnp.exp(x - m), axis=1, keepdims=True)
    a = m + jnp.log(se)
    r = jnp.sum(x, axis=1, keepdims=True)
    xz = x[:, Z:Z + 1]
    cols = jax.lax.broadcasted_iota(jnp.int32, (RB, V), 1)
    xt = jnp.sum(jnp.where(cols == t, x, 0.0), axis=1, keepdims=True)
    lp_t = xt - a
    lp_z = xz - a
    ssum = r - V * a
    loss = kc - s * ssum + (s - c) * lp_t + s * lp_z
    loss = loss + jnp.where(t == Z, s_log_s - s * lp_z, 0.0)
    loss = jnp.where(t == IGNORE_INDEX, 0.0, loss)
    part = jnp.sum(loss, keepdims=True) * (1.0 / B)

    @pl.when(i == 0)
    def _():
        o_ref[...] = jnp.zeros_like(o_ref)

    o_ref[...] += part


def kernel(output, target, one_hot):
    B, V = output.shape
    RB = ROW_BLOCK
    G = B // RB
    t3 = target.reshape(G, RB, 1)
    out = pl.pallas_call(
        functools.partial(_loss_body, V=V, B=B, RB=RB),
        grid=(G,),
        in_specs=[
            pl.BlockSpec((RB, V), lambda i: (i, 0)),
            pl.BlockSpec((1, RB, 1), lambda i: (i, 0, 0)),
        ],
        out_specs=pl.BlockSpec((1, 1), lambda i: (0, 0)),
        out_shape=jax.ShapeDtypeStruct((1, 1), jnp.float32),
    )(output, t3)
    return out[0, 0]
